# baseline (speedup 1.0000x reference)
import functools

import jax
import jax.numpy as jnp
from jax.experimental import pallas as pl

ALPHA = 0.5
BLK = 1000


def _combine_body(x_ref, s1_ref, c1_ref, s2_ref, c2_ref,
                  wst_ref, w1t_ref, w2t_ref, b_ref, o_ref):
    x = x_ref[...]
    a1 = s1_ref[...] / jnp.maximum(c1_ref[...], 1.0)
    a2 = s2_ref[...] / jnp.maximum(c2_ref[...], 1.0)
    out = jnp.dot(x, wst_ref[...], preferred_element_type=jnp.float32)
    out += (1.0 - ALPHA) * jnp.dot(a1, w1t_ref[...],
                                   preferred_element_type=jnp.float32)
    out += ALPHA * jnp.dot(a2, w2t_ref[...],
                           preferred_element_type=jnp.float32)
    o_ref[...] = out + b_ref[...]


@functools.partial(jax.jit, static_argnames=())
def _combine(x, s1, c1, s2, c2, wst, w1t, w2t, bias):
    n, d = x.shape
    grid = (n // BLK,)
    return pl.pallas_call(
        _combine_body,
        grid=grid,
        in_specs=[
            pl.BlockSpec((BLK, d), lambda i: (i, 0)),
            pl.BlockSpec((BLK, d), lambda i: (i, 0)),
            pl.BlockSpec((BLK, 1), lambda i: (i, 0)),
            pl.BlockSpec((BLK, d), lambda i: (i, 0)),
            pl.BlockSpec((BLK, 1), lambda i: (i, 0)),
            pl.BlockSpec((d, d), lambda i: (0, 0)),
            pl.BlockSpec((d, d), lambda i: (0, 0)),
            pl.BlockSpec((d, d), lambda i: (0, 0)),
            pl.BlockSpec((1, d), lambda i: (0, 0)),
        ],
        out_specs=pl.BlockSpec((BLK, d), lambda i: (i, 0)),
        out_shape=jax.ShapeDtypeStruct((n, d), jnp.float32),
    )(x, s1, c1, s2, c2, wst, w1t, w2t, bias)


def kernel(x, edge_index, W_self, b_self, W1, b1, W2, b2):
    src = edge_index[0]
    dst = edge_index[1]
    n = x.shape[0]
    ones = jnp.ones((src.shape[0],), jnp.float32)
    s1 = jax.ops.segment_sum(jnp.take(x, src, axis=0), dst, num_segments=n)
    c1 = jax.ops.segment_sum(ones, dst, num_segments=n)[:, None]
    s2 = jax.ops.segment_sum(jnp.take(x, dst, axis=0), src, num_segments=n)
    c2 = jax.ops.segment_sum(ones, src, num_segments=n)[:, None]
    bias = (b_self + (1.0 - ALPHA) * b1 + ALPHA * b2)[None, :]
    return _combine(x, s1, c1, s2, c2, W_self.T, W1.T, W2.T, bias)


# speedup vs baseline: 2.5369x; 2.5369x over previous
import functools

import jax
import jax.numpy as jnp
from jax import lax
from jax.experimental import pallas as pl
from jax.experimental.pallas import tpu as pltpu
from jax.experimental.pallas import tpu_sc as plsc

ALPHA = 0.5
N = 10000
E = 160000
D = 256
DC = 128
NSUB = 16
NPAD = 10240
STRIPE = NPAD // NSUB
B = 128
NBS = -(--(-E // (NSUB * B)) // 8) * 8
ROWS = NSUB * NBS
EPAD = ROWS * B
CW = 16
IB = 16
BLK = 1000


def _sc_body(src_ref, dst_ref, x0, x1,
             s1_ref, s2_ref, cnt_ref,
             acc, cntacc, gidxs, sidxs, rows1, onesb, zbc, gsem):
    c = lax.axis_index("c")
    s = lax.axis_index("s")
    base = s * NBS
    st = s * STRIPE

    def fill(j, carry):
        onesb[j, pl.ds(0, CW)] = jnp.ones((CW,), jnp.float32)
        zbc[j, pl.ds(0, CW)] = jnp.zeros((CW,), jnp.float32)
        return carry
    lax.fori_loop(0, B, fill, 0)

    def run_pass(gat_hbm, sca_hbm, out_ref, cslot):
        def zfill(j, carry):
            for k in range(DC // 16):
                rows1[j, pl.ds(16 * k, 16)] = jnp.zeros((16,), jnp.float32)
            return carry
        lax.fori_loop(0, B, zfill, 0)
        for r in range(STRIPE // B):
            pltpu.sync_copy(rows1, acc.at[pl.ds(st + r * B, B)])
            pltpu.sync_copy(zbc, cntacc.at[pl.ds(st + r * B, B)])
        plsc.subcore_barrier()

        @pl.when(c == 0)
        def _():
            def chunk(q, carry):
                pltpu.sync_copy(gat_hbm.at[pl.ds(base + q * IB, IB)], gidxs)
                pltpu.sync_copy(sca_hbm.at[pl.ds(base + q * IB, IB)], sidxs)

                def step(j, carry2):
                    pltpu.async_copy(x0.at[gidxs.at[j]], rows1, gsem).wait()
                    pltpu.sync_copy(rows1, acc.at[sidxs.at[j]], add=True)
                    pltpu.sync_copy(onesb, cntacc.at[sidxs.at[j]], add=True)
                    return carry2
                lax.fori_loop(0, IB, step, 0)
                return carry
            lax.fori_loop(0, NBS // IB, chunk, 0)

        @pl.when(c == 1)
        def _():
            def chunk(q, carry):
                pltpu.sync_copy(gat_hbm.at[pl.ds(base + q * IB, IB)], gidxs)
                pltpu.sync_copy(sca_hbm.at[pl.ds(base + q * IB, IB)], sidxs)

                def step(j, carry2):
                    pltpu.async_copy(x1.at[gidxs.at[j]], rows1, gsem).wait()
                    pltpu.sync_copy(rows1, acc.at[sidxs.at[j]], add=True)
                    return carry2
                lax.fori_loop(0, IB, step, 0)
                return carry
            lax.fori_loop(0, NBS // IB, chunk, 0)

        plsc.subcore_barrier()

        @pl.when(c == 0)
        def _():
            pltpu.sync_copy(acc.at[pl.ds(st, STRIPE)],
                            out_ref.at[0, pl.ds(st, STRIPE)])
            pltpu.sync_copy(cntacc.at[pl.ds(st, STRIPE)],
                            cnt_ref.at[cslot, pl.ds(st, STRIPE)])

        @pl.when(c == 1)
        def _():
            pltpu.sync_copy(acc.at[pl.ds(st, STRIPE)],
                            out_ref.at[1, pl.ds(st, STRIPE)])

        plsc.subcore_barrier()

    run_pass(src_ref, dst_ref, s1_ref, 0)
    run_pass(dst_ref, src_ref, s2_ref, 1)


_sc_agg = pl.kernel(
    _sc_body,
    out_type=(
        jax.ShapeDtypeStruct((2, NPAD, DC), jnp.float32),
        jax.ShapeDtypeStruct((2, NPAD, DC), jnp.float32),
        jax.ShapeDtypeStruct((2, NPAD, CW), jnp.float32),
    ),
    mesh=plsc.VectorSubcoreMesh(core_axis_name="c", subcore_axis_name="s"),
    compiler_params=pltpu.CompilerParams(use_tc_tiling_on_sc=False),
    scratch_types=[
        pltpu.VMEM_SHARED((NPAD, DC), jnp.float32),
        pltpu.VMEM_SHARED((NPAD, CW), jnp.float32),
        pltpu.VMEM((IB, B), jnp.int32),
        pltpu.VMEM((IB, B), jnp.int32),
        pltpu.VMEM((B, DC), jnp.float32),
        pltpu.VMEM((B, CW), jnp.float32),
        pltpu.VMEM((B, CW), jnp.float32),
        pltpu.SemaphoreType.DMA,
    ],
)


def _combine_body(x_ref, s1_ref, s2_ref, cnt_ref, wst_ref, w1t_ref, w2t_ref,
                  b_ref, o_ref):
    x = x_ref[...]
    r1 = 1.0 / jnp.maximum(cnt_ref[0, :, 0:1], 1.0)
    r2 = 1.0 / jnp.maximum(cnt_ref[1, :, 0:1], 1.0)
    out = jnp.dot(x, wst_ref[...], preferred_element_type=jnp.float32)
    for t in range(2):
        out += (1.0 - ALPHA) * jnp.dot(
            s1_ref[t] * r1, w1t_ref[t * DC:(t + 1) * DC, :],
            preferred_element_type=jnp.float32)
        out += ALPHA * jnp.dot(
            s2_ref[t] * r2, w2t_ref[t * DC:(t + 1) * DC, :],
            preferred_element_type=jnp.float32)
    o_ref[...] = out + b_ref[...]


def _combine(x, s1, s2, cnt, wst, w1t, w2t, bias):
    n, d = x.shape
    return pl.pallas_call(
        _combine_body,
        grid=(n // BLK,),
        in_specs=[
            pl.BlockSpec((BLK, d), lambda i: (i, 0)),
            pl.BlockSpec((2, BLK, DC), lambda i: (0, i, 0)),
            pl.BlockSpec((2, BLK, DC), lambda i: (0, i, 0)),
            pl.BlockSpec((2, BLK, CW), lambda i: (0, i, 0)),
            pl.BlockSpec((d, d), lambda i: (0, 0)),
            pl.BlockSpec((d, d), lambda i: (0, 0)),
            pl.BlockSpec((d, d), lambda i: (0, 0)),
            pl.BlockSpec((1, d), lambda i: (0, 0)),
        ],
        out_specs=pl.BlockSpec((BLK, d), lambda i: (i, 0)),
        out_shape=jax.ShapeDtypeStruct((n, d), jnp.float32),
    )(x, s1, s2, cnt, wst, w1t, w2t, bias)


def kernel(x, edge_index, W_self, b_self, W1, b1, W2, b2):
    src = edge_index[0].astype(jnp.int32)
    dst = edge_index[1].astype(jnp.int32)
    padi = jnp.full((EPAD - E,), N, jnp.int32)
    src2 = jnp.concatenate([src, padi]).reshape(ROWS, B)
    dst2 = jnp.concatenate([dst, padi]).reshape(ROWS, B)
    xp = jnp.pad(x, ((0, NPAD - N), (0, 0)))
    xcs = [xp[:, t * DC:(t + 1) * DC] for t in range(2)]
    s1, s2, cnt = _sc_agg(src2, dst2, *xcs)
    bias = (b_self + (1.0 - ALPHA) * b1 + ALPHA * b2)[None, :]
    return _combine(x, s1, s2, cnt, W_self.T, W1.T, W2.T, bias)


# speedup vs baseline: 5.0486x; 1.9900x over previous
import functools

import jax
import jax.numpy as jnp
from jax import lax
from jax.experimental import pallas as pl
from jax.experimental.pallas import tpu as pltpu
from jax.experimental.pallas import tpu_sc as plsc

ALPHA = 0.5
N = 10000
E = 160000
D = 256
DC = 128
NSUB = 16
NPAD = 10240
STRIPE = NPAD // NSUB
B = 128
NBS = -(--(-E // (NSUB * B)) // 8) * 8
ROWS = NSUB * NBS
EPAD = ROWS * B
CW = 16
IB = 16
BLK = 1000


def _sc_body(src_ref, dst_ref, x0, x1,
             s1_ref, s2_ref, cnt_ref,
             acc, cntacc, gidxs, sidxs, r0, r1, r2, r3, onesb, zbc,
             gsems, ssems, csem):
    c = lax.axis_index("c")
    s = lax.axis_index("s")
    base = s * NBS
    st = s * STRIPE
    rows = (r0, r1, r2, r3)

    def fill(j, carry):
        onesb[j, pl.ds(0, CW)] = jnp.ones((CW,), jnp.float32)
        zbc[j, pl.ds(0, CW)] = jnp.zeros((CW,), jnp.float32)
        return carry
    lax.fori_loop(0, B, fill, 0)

    def gwait(k):
        pltpu.make_async_copy(x0.at[pl.ds(0, B)], rows[k], gsems.at[k]).wait()

    def swait(k):
        pltpu.make_async_copy(rows[k], acc.at[pl.ds(0, B)], ssems.at[k]).wait()

    def cwait():
        pltpu.make_async_copy(onesb, cntacc.at[pl.ds(0, B)], csem).wait()

    def gissue(j, k):
        @pl.when(c == 0)
        def _():
            pltpu.async_copy(x0.at[gidxs.at[j]], rows[k], gsems.at[k])

        @pl.when(c == 1)
        def _():
            pltpu.async_copy(x1.at[gidxs.at[j]], rows[k], gsems.at[k])

    def run_pass(gat_hbm, sca_hbm, out_ref, pass_id):
        cntp = c == pass_id

        def zfill(j, carry):
            for kk in range(DC // 32):
                r0[j, pl.ds(32 * kk, 32)] = jnp.zeros((32,), jnp.bfloat16)
            return carry
        lax.fori_loop(0, B, zfill, 0)
        for r in range(STRIPE // B):
            pltpu.sync_copy(r0, acc.at[pl.ds(st + r * B, B)])

        @pl.when(cntp)
        def _():
            for r in range(STRIPE // B):
                pltpu.sync_copy(zbc, cntacc.at[pl.ds(st + r * B, B)])

        pltpu.sync_copy(gat_hbm.at[pl.ds(base, NBS)], gidxs)
        pltpu.sync_copy(sca_hbm.at[pl.ds(base, NBS)], sidxs)
        plsc.subcore_barrier()

        for k in range(3):
            gissue(k, k)

        def group(q, carry):
            for k in range(4):
                j = 4 * q + k
                pk = (k + 3) % 4

                @pl.when(j >= 1)
                def _():
                    swait(pk)

                @pl.when(j + 3 < NBS)
                def _():
                    gissue(j + 3, pk)

                gwait(k)
                pltpu.async_copy(rows[k], acc.at[sidxs.at[j]], ssems.at[k],
                                 add=True)

                @pl.when(cntp & (j >= 1))
                def _():
                    cwait()

                @pl.when(cntp)
                def _():
                    pltpu.async_copy(onesb, cntacc.at[sidxs.at[j]], csem,
                                     add=True)
            return carry

        lax.fori_loop(0, NBS // 4, group, 0)
        swait(3)

        @pl.when(cntp)
        def _():
            cwait()

        plsc.subcore_barrier()

        @pl.when(c == 0)
        def _():
            pltpu.sync_copy(acc.at[pl.ds(st, STRIPE)],
                            out_ref.at[0, pl.ds(st, STRIPE)])

        @pl.when(c == 1)
        def _():
            pltpu.sync_copy(acc.at[pl.ds(st, STRIPE)],
                            out_ref.at[1, pl.ds(st, STRIPE)])

        @pl.when(cntp)
        def _():
            pltpu.sync_copy(cntacc.at[pl.ds(st, STRIPE)],
                            cnt_ref.at[pass_id, pl.ds(st, STRIPE)])

        plsc.subcore_barrier()

    run_pass(src_ref, dst_ref, s1_ref, 0)
    run_pass(dst_ref, src_ref, s2_ref, 1)


_sc_agg = pl.kernel(
    _sc_body,
    out_type=(
        jax.ShapeDtypeStruct((2, NPAD, DC), jnp.bfloat16),
        jax.ShapeDtypeStruct((2, NPAD, DC), jnp.bfloat16),
        jax.ShapeDtypeStruct((2, NPAD, CW), jnp.float32),
    ),
    mesh=plsc.VectorSubcoreMesh(core_axis_name="c", subcore_axis_name="s"),
    compiler_params=pltpu.CompilerParams(use_tc_tiling_on_sc=False),
    scratch_types=[
        pltpu.VMEM_SHARED((NPAD, DC), jnp.bfloat16),
        pltpu.VMEM_SHARED((NPAD, CW), jnp.float32),
        pltpu.VMEM((NBS, B), jnp.int32),
        pltpu.VMEM((NBS, B), jnp.int32),
        pltpu.VMEM((B, DC), jnp.bfloat16),
        pltpu.VMEM((B, DC), jnp.bfloat16),
        pltpu.VMEM((B, DC), jnp.bfloat16),
        pltpu.VMEM((B, DC), jnp.bfloat16),
        pltpu.VMEM((B, CW), jnp.float32),
        pltpu.VMEM((B, CW), jnp.float32),
        pltpu.SemaphoreType.DMA((4,)),
        pltpu.SemaphoreType.DMA((4,)),
        pltpu.SemaphoreType.DMA,
    ],
)


def _combine_body(x_ref, s1_ref, s2_ref, cnt_ref, wst_ref, w1t_ref, w2t_ref,
                  b_ref, o_ref):
    x = x_ref[...]
    r1 = 1.0 / jnp.maximum(cnt_ref[0, :, 0:1], 1.0)
    r2 = 1.0 / jnp.maximum(cnt_ref[1, :, 0:1], 1.0)
    out = jnp.dot(x, wst_ref[...], preferred_element_type=jnp.float32)
    for t in range(2):
        out += (1.0 - ALPHA) * jnp.dot(
            s1_ref[t].astype(jnp.float32) * r1, w1t_ref[t * DC:(t + 1) * DC, :],
            preferred_element_type=jnp.float32)
        out += ALPHA * jnp.dot(
            s2_ref[t].astype(jnp.float32) * r2, w2t_ref[t * DC:(t + 1) * DC, :],
            preferred_element_type=jnp.float32)
    o_ref[...] = out + b_ref[...]


def _combine(x, s1, s2, cnt, wst, w1t, w2t, bias):
    n, d = x.shape
    return pl.pallas_call(
        _combine_body,
        grid=(n // BLK,),
        in_specs=[
            pl.BlockSpec((BLK, d), lambda i: (i, 0)),
            pl.BlockSpec((2, BLK, DC), lambda i: (0, i, 0)),
            pl.BlockSpec((2, BLK, DC), lambda i: (0, i, 0)),
            pl.BlockSpec((2, BLK, CW), lambda i: (0, i, 0)),
            pl.BlockSpec((d, d), lambda i: (0, 0)),
            pl.BlockSpec((d, d), lambda i: (0, 0)),
            pl.BlockSpec((d, d), lambda i: (0, 0)),
            pl.BlockSpec((1, d), lambda i: (0, 0)),
        ],
        out_specs=pl.BlockSpec((BLK, d), lambda i: (i, 0)),
        out_shape=jax.ShapeDtypeStruct((n, d), jnp.float32),
    )(x, s1, s2, cnt, wst, w1t, w2t, bias)


def kernel(x, edge_index, W_self, b_self, W1, b1, W2, b2):
    src = edge_index[0].astype(jnp.int32)
    dst = edge_index[1].astype(jnp.int32)
    padi = jnp.full((EPAD - E,), N, jnp.int32)
    src2 = jnp.concatenate([src, padi]).reshape(ROWS, B)
    dst2 = jnp.concatenate([dst, padi]).reshape(ROWS, B)
    xp = jnp.pad(x.astype(jnp.bfloat16), ((0, NPAD - N), (0, 0)))
    xcs = [xp[:, t * DC:(t + 1) * DC] for t in range(2)]
    s1, s2, cnt = _sc_agg(src2, dst2, *xcs)
    bias = (b_self + (1.0 - ALPHA) * b1 + ALPHA * b2)[None, :]
    return _combine(x, s1, s2, cnt, W_self.T, W1.T, W2.T, bias)


# speedup vs baseline: 6.2943x; 1.2468x over previous
import functools

import jax
import jax.numpy as jnp
from jax import lax
from jax.experimental import pallas as pl
from jax.experimental.pallas import tpu as pltpu
from jax.experimental.pallas import tpu_sc as plsc

ALPHA = 0.5
N = 10000
E = 160000
D = 256
DC = 128
NSUB = 16
NPAD = 10240
STRIPE = NPAD // NSUB
B = 128
NBS = -(--(-E // (NSUB * B)) // 8) * 8
ROWS = NSUB * NBS
EPAD = ROWS * B
CW = 16
IB = 16
BLK = 1000


def _sc_body(src_ref, dst_ref, x0, x1,
             s1_ref, s2_ref, cnt_ref,
             acc, cntacc, gidxs, sidxs, r0, r1, r2, r3, onesb, zbc,
             gsems, ssems, csem):
    c = lax.axis_index("c")
    s = lax.axis_index("s")
    base = s * NBS
    st = s * STRIPE
    rows = (r0, r1, r2, r3)

    def fill(j, carry):
        onesb[j, pl.ds(0, CW)] = jnp.ones((CW,), jnp.float32)
        zbc[j, pl.ds(0, CW)] = jnp.zeros((CW,), jnp.float32)
        return carry
    lax.fori_loop(0, B, fill, 0)

    def gwait(k):
        pltpu.make_async_copy(x0.at[pl.ds(0, B)], rows[k], gsems.at[k]).wait()

    def swait(k):
        pltpu.make_async_copy(rows[k], acc.at[pl.ds(0, B)], ssems.at[k]).wait()

    def cwait():
        pltpu.make_async_copy(onesb, cntacc.at[pl.ds(0, B)], csem).wait()

    def gissue(j, k):
        @pl.when(c == 0)
        def _():
            pltpu.async_copy(x0.at[gidxs.at[j]], rows[k], gsems.at[k])

        @pl.when(c == 1)
        def _():
            pltpu.async_copy(x1.at[gidxs.at[j]], rows[k], gsems.at[k])

    def run_pass(gat_hbm, sca_hbm, out_ref, pass_id):
        cntp = c == pass_id

        def zfill(j, carry):
            for kk in range(DC // 32):
                r0[j, pl.ds(32 * kk, 32)] = jnp.zeros((32,), jnp.bfloat16)
            return carry
        lax.fori_loop(0, B, zfill, 0)
        for r in range(STRIPE // B):
            pltpu.sync_copy(r0, acc.at[pl.ds(st + r * B, B)])

        @pl.when(cntp)
        def _():
            for r in range(STRIPE // B):
                pltpu.sync_copy(zbc, cntacc.at[pl.ds(st + r * B, B)])

        pltpu.sync_copy(gat_hbm.at[pl.ds(base, NBS)], gidxs)
        pltpu.sync_copy(sca_hbm.at[pl.ds(base, NBS)], sidxs)
        plsc.subcore_barrier()

        for k in range(3):
            gissue(k, k)

        def group(q, carry):
            for k in range(4):
                j = 4 * q + k
                pk = (k + 3) % 4

                @pl.when(j >= 1)
                def _():
                    swait(pk)

                @pl.when(j + 3 < NBS)
                def _():
                    gissue(j + 3, pk)

                gwait(k)
                pltpu.async_copy(rows[k], acc.at[sidxs.at[j]], ssems.at[k],
                                 add=True)

                @pl.when(cntp & (j >= 1))
                def _():
                    cwait()

                @pl.when(cntp)
                def _():
                    pltpu.async_copy(onesb, cntacc.at[sidxs.at[j]], csem,
                                     add=True)
            return carry

        lax.fori_loop(0, NBS // 4, group, 0)
        swait(3)

        @pl.when(cntp)
        def _():
            cwait()

        plsc.subcore_barrier()

        @pl.when(c == 0)
        def _():
            pltpu.sync_copy(acc.at[pl.ds(st, STRIPE)],
                            out_ref.at[0, pl.ds(st, STRIPE)])

        @pl.when(c == 1)
        def _():
            pltpu.sync_copy(acc.at[pl.ds(st, STRIPE)],
                            out_ref.at[1, pl.ds(st, STRIPE)])

        @pl.when(cntp)
        def _():
            pltpu.sync_copy(cntacc.at[pl.ds(st, STRIPE)],
                            cnt_ref.at[pass_id, pl.ds(st, STRIPE)])

        plsc.subcore_barrier()

    run_pass(src_ref, dst_ref, s1_ref, 0)
    run_pass(dst_ref, src_ref, s2_ref, 1)


_sc_agg = pl.kernel(
    _sc_body,
    out_type=(
        jax.ShapeDtypeStruct((2, NPAD, DC), jnp.bfloat16),
        jax.ShapeDtypeStruct((2, NPAD, DC), jnp.bfloat16),
        jax.ShapeDtypeStruct((2, NPAD, CW), jnp.float32),
    ),
    mesh=plsc.VectorSubcoreMesh(core_axis_name="c", subcore_axis_name="s"),
    compiler_params=pltpu.CompilerParams(use_tc_tiling_on_sc=False),
    scratch_types=[
        pltpu.VMEM_SHARED((NPAD, DC), jnp.bfloat16),
        pltpu.VMEM_SHARED((NPAD, CW), jnp.float32),
        pltpu.VMEM((NBS, B), jnp.int32),
        pltpu.VMEM((NBS, B), jnp.int32),
        pltpu.VMEM((B, DC), jnp.bfloat16),
        pltpu.VMEM((B, DC), jnp.bfloat16),
        pltpu.VMEM((B, DC), jnp.bfloat16),
        pltpu.VMEM((B, DC), jnp.bfloat16),
        pltpu.VMEM((B, CW), jnp.float32),
        pltpu.VMEM((B, CW), jnp.float32),
        pltpu.SemaphoreType.DMA((4,)),
        pltpu.SemaphoreType.DMA((4,)),
        pltpu.SemaphoreType.DMA,
    ],
)


def _combine_body(x0_ref, x1_ref, s1_ref, s2_ref, cnt_ref,
                  wst_ref, w1t_ref, w2t_ref, b_ref, o_ref):
    r1 = 1.0 / jnp.maximum(cnt_ref[0, :, 0:1], 1.0)
    r2 = 1.0 / jnp.maximum(cnt_ref[1, :, 0:1], 1.0)
    xs = (x0_ref, x1_ref)
    out = b_ref[...]
    m1 = jnp.zeros_like(out)
    m2 = jnp.zeros_like(out)
    for t in range(2):
        w = slice(t * DC, (t + 1) * DC)
        out += jnp.dot(xs[t][...], wst_ref[w, :],
                       preferred_element_type=jnp.float32)
        m1 += jnp.dot(s1_ref[t], w1t_ref[w, :],
                      preferred_element_type=jnp.float32)
        m2 += jnp.dot(s2_ref[t], w2t_ref[w, :],
                      preferred_element_type=jnp.float32)
    o_ref[...] = out + (1.0 - ALPHA) * r1 * m1 + ALPHA * r2 * m2


def _combine(x0, x1, s1, s2, cnt, wst, w1t, w2t, bias):
    n, d = NPAD, D
    return pl.pallas_call(
        _combine_body,
        grid=(N // BLK,),
        in_specs=[
            pl.BlockSpec((BLK, DC), lambda i: (i, 0)),
            pl.BlockSpec((BLK, DC), lambda i: (i, 0)),
            pl.BlockSpec((2, BLK, DC), lambda i: (0, i, 0)),
            pl.BlockSpec((2, BLK, DC), lambda i: (0, i, 0)),
            pl.BlockSpec((2, BLK, CW), lambda i: (0, i, 0)),
            pl.BlockSpec((d, d), lambda i: (0, 0)),
            pl.BlockSpec((d, d), lambda i: (0, 0)),
            pl.BlockSpec((d, d), lambda i: (0, 0)),
            pl.BlockSpec((1, d), lambda i: (0, 0)),
        ],
        out_specs=pl.BlockSpec((BLK, d), lambda i: (i, 0)),
        out_shape=jax.ShapeDtypeStruct((N, d), jnp.float32),
    )(x0, x1, s1, s2, cnt, wst, w1t, w2t, bias)


def kernel(x, edge_index, W_self, b_self, W1, b1, W2, b2):
    src = edge_index[0].astype(jnp.int32)
    dst = edge_index[1].astype(jnp.int32)
    padi = jnp.full((EPAD - E,), N, jnp.int32)
    src2 = jnp.concatenate([src, padi]).reshape(ROWS, B)
    dst2 = jnp.concatenate([dst, padi]).reshape(ROWS, B)
    xp = jnp.pad(x.astype(jnp.bfloat16), ((0, NPAD - N), (0, 0)))
    xcs = [xp[:, t * DC:(t + 1) * DC] for t in range(2)]
    s1, s2, cnt = _sc_agg(src2, dst2, *xcs)
    bias = (b_self + (1.0 - ALPHA) * b1 + ALPHA * b2)[None, :]
    bf = jnp.bfloat16
    return _combine(xcs[0], xcs[1], s1, s2, cnt, W_self.T.astype(bf),
                    W1.T.astype(bf), W2.T.astype(bf), bias)


# speedup vs baseline: 6.2972x; 1.0005x over previous
import functools

import jax
import jax.numpy as jnp
from jax import lax
from jax.experimental import pallas as pl
from jax.experimental.pallas import tpu as pltpu
from jax.experimental.pallas import tpu_sc as plsc

ALPHA = 0.5
N = 10000
E = 160000
D = 256
DC = 128
NSUB = 16
NPAD = 10240
STRIPE = NPAD // NSUB
B = 128
NBS = -(--(-E // (NSUB * B)) // 8) * 8
ROWS = NSUB * NBS
EPAD = ROWS * B
CW = 16
NB = 5
IB = 16
BLK = 1000


def _sc_body(src_ref, dst_ref, x0, x1,
             s1_ref, s2_ref, cnt_ref,
             acc, cntacc, gidxs, sidxs, r0, r1, r2, r3, r4, onesb, zbc,
             gsems, ssems, csem):
    c = lax.axis_index("c")
    s = lax.axis_index("s")
    base = s * NBS
    st = s * STRIPE
    rows = (r0, r1, r2, r3, r4)

    def fill(j, carry):
        onesb[j, pl.ds(0, CW)] = jnp.ones((CW,), jnp.float32)
        zbc[j, pl.ds(0, CW)] = jnp.zeros((CW,), jnp.float32)
        return carry
    lax.fori_loop(0, B, fill, 0)

    def gwait(k):
        pltpu.make_async_copy(x0.at[pl.ds(0, B)], rows[k], gsems.at[k]).wait()

    def swait(k):
        pltpu.make_async_copy(rows[k], acc.at[pl.ds(0, B)], ssems.at[k]).wait()

    def cwait():
        pltpu.make_async_copy(onesb, cntacc.at[pl.ds(0, B)], csem).wait()

    def gissue(j, k):
        @pl.when(c == 0)
        def _():
            pltpu.async_copy(x0.at[gidxs.at[j]], rows[k], gsems.at[k])

        @pl.when(c == 1)
        def _():
            pltpu.async_copy(x1.at[gidxs.at[j]], rows[k], gsems.at[k])

    def run_pass(gat_hbm, sca_hbm, out_ref, pass_id):
        cntp = c == pass_id

        def zfill(j, carry):
            for kk in range(DC // 32):
                r0[j, pl.ds(32 * kk, 32)] = jnp.zeros((32,), jnp.bfloat16)
            return carry
        lax.fori_loop(0, B, zfill, 0)
        for r in range(STRIPE // B):
            pltpu.sync_copy(r0, acc.at[pl.ds(st + r * B, B)])

        @pl.when(cntp)
        def _():
            for r in range(STRIPE // B):
                pltpu.sync_copy(zbc, cntacc.at[pl.ds(st + r * B, B)])

        pltpu.sync_copy(gat_hbm.at[pl.ds(base, NBS)], gidxs)
        pltpu.sync_copy(sca_hbm.at[pl.ds(base, NBS)], sidxs)
        plsc.subcore_barrier()

        for k in range(3):
            gissue(k, k)

        def group(q, carry):
            for k in range(NB):
                j = NB * q + k
                pk = (k + 3) % NB

                @pl.when(j >= 2)
                def _():
                    swait(pk)

                @pl.when(j + 3 < NBS)
                def _():
                    gissue(j + 3, pk)

                gwait(k)
                pltpu.async_copy(rows[k], acc.at[sidxs.at[j]], ssems.at[k],
                                 add=True)

                @pl.when(cntp & (j >= 1))
                def _():
                    cwait()

                @pl.when(cntp)
                def _():
                    pltpu.async_copy(onesb, cntacc.at[sidxs.at[j]], csem,
                                     add=True)
            return carry

        lax.fori_loop(0, NBS // NB, group, 0)
        swait((NBS - 2) % NB)
        swait((NBS - 1) % NB)

        @pl.when(cntp)
        def _():
            cwait()

        plsc.subcore_barrier()

        @pl.when(c == 0)
        def _():
            pltpu.sync_copy(acc.at[pl.ds(st, STRIPE)],
                            out_ref.at[0, pl.ds(st, STRIPE)])

        @pl.when(c == 1)
        def _():
            pltpu.sync_copy(acc.at[pl.ds(st, STRIPE)],
                            out_ref.at[1, pl.ds(st, STRIPE)])

        @pl.when(cntp)
        def _():
            pltpu.sync_copy(cntacc.at[pl.ds(st, STRIPE)],
                            cnt_ref.at[pass_id, pl.ds(st, STRIPE)])

        plsc.subcore_barrier()

    run_pass(src_ref, dst_ref, s1_ref, 0)
    run_pass(dst_ref, src_ref, s2_ref, 1)


_sc_agg = pl.kernel(
    _sc_body,
    out_type=(
        jax.ShapeDtypeStruct((2, NPAD, DC), jnp.bfloat16),
        jax.ShapeDtypeStruct((2, NPAD, DC), jnp.bfloat16),
        jax.ShapeDtypeStruct((2, NPAD, CW), jnp.float32),
    ),
    mesh=plsc.VectorSubcoreMesh(core_axis_name="c", subcore_axis_name="s"),
    compiler_params=pltpu.CompilerParams(use_tc_tiling_on_sc=False),
    scratch_types=[
        pltpu.VMEM_SHARED((NPAD, DC), jnp.bfloat16),
        pltpu.VMEM_SHARED((NPAD, CW), jnp.float32),
        pltpu.VMEM((NBS, B), jnp.int32),
        pltpu.VMEM((NBS, B), jnp.int32),
        pltpu.VMEM((B, DC), jnp.bfloat16),
        pltpu.VMEM((B, DC), jnp.bfloat16),
        pltpu.VMEM((B, DC), jnp.bfloat16),
        pltpu.VMEM((B, DC), jnp.bfloat16),
        pltpu.VMEM((B, DC), jnp.bfloat16),
        pltpu.VMEM((B, CW), jnp.float32),
        pltpu.VMEM((B, CW), jnp.float32),
        pltpu.SemaphoreType.DMA((NB,)),
        pltpu.SemaphoreType.DMA((NB,)),
        pltpu.SemaphoreType.DMA,
    ],
)


def _combine_body(x0_ref, x1_ref, s1_ref, s2_ref, cnt_ref,
                  wst_ref, w1t_ref, w2t_ref, b_ref, o_ref):
    r1 = 1.0 / jnp.maximum(cnt_ref[0, :, 0:1], 1.0)
    r2 = 1.0 / jnp.maximum(cnt_ref[1, :, 0:1], 1.0)
    xs = (x0_ref, x1_ref)
    out = b_ref[...]
    m1 = jnp.zeros_like(out)
    m2 = jnp.zeros_like(out)
    for t in range(2):
        w = slice(t * DC, (t + 1) * DC)
        out += jnp.dot(xs[t][...], wst_ref[w, :],
                       preferred_element_type=jnp.float32)
        m1 += jnp.dot(s1_ref[t], w1t_ref[w, :],
                      preferred_element_type=jnp.float32)
        m2 += jnp.dot(s2_ref[t], w2t_ref[w, :],
                      preferred_element_type=jnp.float32)
    o_ref[...] = out + (1.0 - ALPHA) * r1 * m1 + ALPHA * r2 * m2


def _combine(x0, x1, s1, s2, cnt, wst, w1t, w2t, bias):
    n, d = NPAD, D
    return pl.pallas_call(
        _combine_body,
        grid=(N // BLK,),
        in_specs=[
            pl.BlockSpec((BLK, DC), lambda i: (i, 0)),
            pl.BlockSpec((BLK, DC), lambda i: (i, 0)),
            pl.BlockSpec((2, BLK, DC), lambda i: (0, i, 0)),
            pl.BlockSpec((2, BLK, DC), lambda i: (0, i, 0)),
            pl.BlockSpec((2, BLK, CW), lambda i: (0, i, 0)),
            pl.BlockSpec((d, d), lambda i: (0, 0)),
            pl.BlockSpec((d, d), lambda i: (0, 0)),
            pl.BlockSpec((d, d), lambda i: (0, 0)),
            pl.BlockSpec((1, d), lambda i: (0, 0)),
        ],
        out_specs=pl.BlockSpec((BLK, d), lambda i: (i, 0)),
        out_shape=jax.ShapeDtypeStruct((N, d), jnp.float32),
    )(x0, x1, s1, s2, cnt, wst, w1t, w2t, bias)


def kernel(x, edge_index, W_self, b_self, W1, b1, W2, b2):
    src = edge_index[0].astype(jnp.int32)
    dst = edge_index[1].astype(jnp.int32)
    padi = jnp.full((EPAD - E,), N, jnp.int32)
    src2 = jnp.concatenate([src, padi]).reshape(ROWS, B)
    dst2 = jnp.concatenate([dst, padi]).reshape(ROWS, B)
    xp = jnp.pad(x.astype(jnp.bfloat16), ((0, NPAD - N), (0, 0)))
    xcs = [xp[:, t * DC:(t + 1) * DC] for t in range(2)]
    s1, s2, cnt = _sc_agg(src2, dst2, *xcs)
    bias = (b_self + (1.0 - ALPHA) * b1 + ALPHA * b2)[None, :]
    bf = jnp.bfloat16
    return _combine(xcs[0], xcs[1], s1, s2, cnt, W_self.T.astype(bf),
                    W1.T.astype(bf), W2.T.astype(bf), bias)
